# baseline (speedup 1.0000x reference)
import jax
import jax.numpy as jnp
from jax.experimental import pallas as pl
from jax.experimental.pallas import tpu as pltpu

_H = 32
_W = 32
_D = 256
_B = 8


def _body(row_ref, col_ref, out_hbm, pos_ref, sem):
    ce = col_ref[:_W, :]
    re = row_ref[:_H, :]
    pos_ref[:, :, :_D] = jnp.broadcast_to(ce[None, :, :], (_H, _W, _D))
    pos_ref[:, :, _D:] = jnp.broadcast_to(re[:, None, :], (_H, _W, _D))
    c0 = pltpu.make_async_copy(pos_ref, out_hbm.at[0], sem.at[0])
    c0.start()
    c0.wait()
    copies = [
        pltpu.make_async_copy(pos_ref, out_hbm.at[b], sem.at[b])
        for b in range(1, 5)
    ] + [
        pltpu.make_async_copy(out_hbm.at[0], out_hbm.at[b], sem.at[b])
        for b in range(5, _B)
    ]
    for c in copies:
        c.start()
    for c in copies:
        c.wait()


def kernel(x, row_embed, col_embed):
    b = x.shape[0]
    out = pl.pallas_call(
        _body,
        in_specs=[
            pl.BlockSpec(memory_space=pltpu.MemorySpace.VMEM),
            pl.BlockSpec(memory_space=pltpu.MemorySpace.VMEM),
        ],
        out_specs=pl.BlockSpec(memory_space=pltpu.MemorySpace.HBM),
        out_shape=jax.ShapeDtypeStruct((b, _H, _W, 2 * _D), jnp.float32),
        scratch_shapes=[
            pltpu.VMEM((_H, _W, 2 * _D), jnp.float32),
            pltpu.SemaphoreType.DMA((_B,)),
        ],
    )(row_embed, col_embed)
    return out.transpose(0, 3, 1, 2)


# speedup vs baseline: 27.5378x; 27.5378x over previous
import jax
import jax.numpy as jnp
from jax.experimental import pallas as pl
from jax.experimental.pallas import tpu as pltpu

_H = 32
_W = 32
_D = 256
_B = 8


def _body(row_ref, col_ref, out_hbm, pos_ref, sem):
    ce = col_ref[:_W, :]
    re = row_ref[:_H, :]
    hh = _H // 2
    pos_ref[:hh, :, :_D] = jnp.broadcast_to(ce[None, :, :], (hh, _W, _D))
    pos_ref[:hh, :, _D:] = jnp.broadcast_to(re[:hh, None, :], (hh, _W, _D))
    first = [
        pltpu.make_async_copy(
            pos_ref.at[pl.ds(0, hh)], out_hbm.at[b, pl.ds(0, hh)], sem.at[b])
        for b in range(_B)
    ]
    for c in first:
        c.start()
    pos_ref[hh:, :, :_D] = jnp.broadcast_to(ce[None, :, :], (hh, _W, _D))
    pos_ref[hh:, :, _D:] = jnp.broadcast_to(re[hh:, None, :], (hh, _W, _D))
    second = [
        pltpu.make_async_copy(
            pos_ref.at[pl.ds(hh, hh)], out_hbm.at[b, pl.ds(hh, hh)],
            sem.at[_B + b])
        for b in range(_B)
    ]
    for c in second:
        c.start()
    for c in first + second:
        c.wait()


def kernel(x, row_embed, col_embed):
    b = x.shape[0]
    out = pl.pallas_call(
        _body,
        in_specs=[
            pl.BlockSpec(memory_space=pltpu.MemorySpace.VMEM),
            pl.BlockSpec(memory_space=pltpu.MemorySpace.VMEM),
        ],
        out_specs=pl.BlockSpec(memory_space=pltpu.MemorySpace.HBM),
        out_shape=jax.ShapeDtypeStruct((b, _H, _W, 2 * _D), jnp.float32),
        scratch_shapes=[
            pltpu.VMEM((_H, _W, 2 * _D), jnp.float32),
            pltpu.SemaphoreType.DMA((2 * _B,)),
        ],
    )(row_embed, col_embed)
    return out.transpose(0, 3, 1, 2)
